# baseline (speedup 1.0000x reference)
import functools

import jax
import jax.numpy as jnp
from jax.experimental import pallas as pl


def _tree_body(depth, B, D, H, R,
               nf_ref, rel_ref, rel_emb_ref, watt_ref, Wf_ref, bf_ref,
               Wix_ref, Wih_ref, bi_ref,
               Wox_ref, Woh_ref, bo_ref,
               Wux_ref, Wuh_ref, bu_ref,
               out_ref):
    f32 = jnp.float32
    bf16 = jnp.bfloat16

    def mm(a, b):
        return jnp.dot(a.astype(bf16), b, preferred_element_type=f32)

    n = 2 ** (depth - 1)
    xf = nf_ref[n - 1:2 * n - 1].reshape(n * B, D)
    zi = mm(xf, Wix_ref[...]) + bi_ref[...]
    zo = mm(xf, Wox_ref[...]) + bo_ref[...]
    zu = mm(xf, Wux_ref[...]) + bu_ref[...]
    c = jax.nn.sigmoid(zi) * jnp.tanh(zu)
    h = jax.nn.sigmoid(zo) * jnp.tanh(c)

    watt = watt_ref[...].reshape(1, 1, H)

    for l in range(depth - 2, -1, -1):
        n = 2 ** l
        m = 2 * n
        c0 = 2 * n - 1

        h3 = h.reshape(m, B, H)
        c3 = c.reshape(m, B, H)

        rel = rel_ref[c0:c0 + m]
        masks = [(rel == r).astype(f32) for r in range(R)]
        remb = masks[0] * rel_emb_ref[0:1, :].reshape(1, 1, H)
        bfs = masks[0] * bf_ref[0:1, :].reshape(1, 1, H)
        for r in range(1, R):
            remb = remb + masks[r] * rel_emb_ref[r:r + 1, :].reshape(1, 1, H)
            bfs = bfs + masks[r] * bf_ref[r:r + 1, :].reshape(1, 1, H)

        s = jnp.sum((h3 + remb) * watt, axis=-1, keepdims=True)
        s4 = s.reshape(n, 2, B, 1)
        a = jax.nn.sigmoid(s4[:, 0] - s4[:, 1])
        h4 = h3.reshape(n, 2, B, H)
        h_sum = a * h4[:, 0] + (1.0 - a) * h4[:, 1]

        hb = h.astype(bf16)
        f = jnp.broadcast_to(bfs, (m, B, H))
        for r in range(R):
            fr = mm(hb, Wf_ref[r]).reshape(m, B, H)
            f = f + masks[r] * fr
        fc = (f * c3).reshape(n, 2, B, H)
        c_sum = (fc[:, 0] + fc[:, 1]).reshape(n * B, H)

        xf = nf_ref[n - 1:2 * n - 1].reshape(n * B, D)
        hs = h_sum.reshape(n * B, H).astype(bf16)
        zi = mm(xf, Wix_ref[...]) + mm(hs, Wih_ref[...]) + bi_ref[...]
        zo = mm(xf, Wox_ref[...]) + mm(hs, Woh_ref[...]) + bo_ref[...]
        zu = mm(xf, Wux_ref[...]) + mm(hs, Wuh_ref[...]) + bu_ref[...]
        c = jax.nn.sigmoid(zi) * jnp.tanh(zu) + c_sum
        h = jax.nn.sigmoid(zo) * jnp.tanh(c)

    out_ref[...] = h


def kernel(node_features, rel_emb, W_att, b_att, W_i, b_i, W_o, b_o,
           W_u, b_u, Wf, bf, W_enc, b_enc, relation_ids):
    B, N, D = node_features.shape
    R, H = rel_emb.shape
    depth = (N + 1).bit_length() - 1

    bf16 = jnp.bfloat16
    nf = jnp.transpose(node_features, (1, 0, 2)).astype(bf16)
    relH = jnp.broadcast_to(
        relation_ids.astype(jnp.int32)[:, None, None], (N, 1, H))
    watt = W_att.reshape(1, H)

    body = functools.partial(_tree_body, depth, B, D, H, R)
    return pl.pallas_call(
        body,
        out_shape=jax.ShapeDtypeStruct((B, H), jnp.float32),
    )(nf, relH, rel_emb, watt, Wf.astype(bf16), bf,
      W_i[:D].astype(bf16), W_i[D:].astype(bf16), b_i.reshape(1, H),
      W_o[:D].astype(bf16), W_o[D:].astype(bf16), b_o.reshape(1, H),
      W_u[:D].astype(bf16), W_u[D:].astype(bf16), b_u.reshape(1, H))


# speedup vs baseline: 1.0897x; 1.0897x over previous
import functools

import jax
import jax.numpy as jnp
from jax.experimental import pallas as pl
from jax.experimental.pallas import tpu as pltpu


def _tree_body(depth, B, D, H, R,
               nf_ref, rel_ref, rel_emb_ref, watt_ref, Wf_ref, bf_ref,
               Wix_ref, Wih_ref, bi_ref,
               Wox_ref, Woh_ref, bo_ref,
               Wux_ref, Wuh_ref, bu_ref,
               out_ref):
    f32 = jnp.float32

    def mm(a, b):
        return jnp.dot(a, b, preferred_element_type=f32)

    n = 2 ** (depth - 1)
    xf = nf_ref[n - 1:2 * n - 1].reshape(n * B, D)
    zi = mm(xf, Wix_ref[...]) + bi_ref[...]
    zo = mm(xf, Wox_ref[...]) + bo_ref[...]
    zu = mm(xf, Wux_ref[...]) + bu_ref[...]
    c = jax.nn.sigmoid(zi) * jnp.tanh(zu)
    h = jax.nn.sigmoid(zo) * jnp.tanh(c)

    watt = watt_ref[...].reshape(1, 1, H)

    for l in range(depth - 2, -1, -1):
        n = 2 ** l
        m = 2 * n
        c0 = 2 * n - 1

        h3 = h.reshape(m, B, H)
        c3 = c.reshape(m, B, H)

        rel = rel_ref[c0:c0 + m]
        masks = [(rel == r).astype(f32) for r in range(R)]
        remb = masks[0] * rel_emb_ref[0:1, :].reshape(1, 1, H)
        bfs = masks[0] * bf_ref[0:1, :].reshape(1, 1, H)
        for r in range(1, R):
            remb = remb + masks[r] * rel_emb_ref[r:r + 1, :].reshape(1, 1, H)
            bfs = bfs + masks[r] * bf_ref[r:r + 1, :].reshape(1, 1, H)

        s = jnp.sum((h3 + remb) * watt, axis=-1, keepdims=True)
        s4 = s.reshape(n, 2, B, 1)
        a = jax.nn.sigmoid(s4[:, 0] - s4[:, 1])
        h4 = h3.reshape(n, 2, B, H)
        h_sum = a * h4[:, 0] + (1.0 - a) * h4[:, 1]

        f = jnp.broadcast_to(bfs, (m, B, H))
        for r in range(R):
            fr = mm(h, Wf_ref[r]).reshape(m, B, H)
            f = f + masks[r] * fr
        fc = (f * c3).reshape(n, 2, B, H)
        c_sum = (fc[:, 0] + fc[:, 1]).reshape(n * B, H)

        xf = nf_ref[n - 1:2 * n - 1].reshape(n * B, D)
        hs = h_sum.reshape(n * B, H)
        zi = mm(xf, Wix_ref[...]) + mm(hs, Wih_ref[...]) + bi_ref[...]
        zo = mm(xf, Wox_ref[...]) + mm(hs, Woh_ref[...]) + bo_ref[...]
        zu = mm(xf, Wux_ref[...]) + mm(hs, Wuh_ref[...]) + bu_ref[...]
        c = jax.nn.sigmoid(zi) * jnp.tanh(zu) + c_sum
        h = jax.nn.sigmoid(zo) * jnp.tanh(c)

    out_ref[...] = h


def kernel(node_features, rel_emb, W_att, b_att, W_i, b_i, W_o, b_o,
           W_u, b_u, Wf, bf, W_enc, b_enc, relation_ids):
    B, N, D = node_features.shape
    R, H = rel_emb.shape
    depth = (N + 1).bit_length() - 1

    nf = jnp.transpose(node_features, (1, 0, 2))
    relH = jnp.broadcast_to(
        relation_ids.astype(jnp.int32)[:, None, None], (N, 1, H))
    watt = W_att.reshape(1, H)

    G = 2
    Bp = B // G
    body = functools.partial(_tree_body, depth, Bp, D, H, R)

    def full(shape):
        return pl.BlockSpec(shape, lambda i: (0,) * len(shape))

    return pl.pallas_call(
        body,
        grid=(G,),
        in_specs=[
            pl.BlockSpec((N, Bp, D), lambda i: (0, i, 0)),
            full((N, 1, H)),
            full((R, H)),
            full((1, H)),
            full((R, H, H)),
            full((R, H)),
            full((D, H)), full((H, H)), full((1, H)),
            full((D, H)), full((H, H)), full((1, H)),
            full((D, H)), full((H, H)), full((1, H)),
        ],
        out_specs=pl.BlockSpec((Bp, H), lambda i: (i, 0)),
        out_shape=jax.ShapeDtypeStruct((B, H), jnp.float32),
        compiler_params=pltpu.CompilerParams(
            dimension_semantics=("parallel",)),
    )(nf, relH, rel_emb, watt, Wf, bf,
      W_i[:D], W_i[D:], b_i.reshape(1, H),
      W_o[:D], W_o[D:], b_o.reshape(1, H),
      W_u[:D], W_u[D:], b_u.reshape(1, H))


# speedup vs baseline: 1.1722x; 1.0757x over previous
import functools

import jax
import jax.numpy as jnp
from jax.experimental import pallas as pl


def _tree_body(depth, B, D, H, R,
               nf_ref, rel_ref, rel_emb_ref, watt_ref, Wfcat_ref, bf_ref,
               Wxcat_ref, Whcat_ref, bcat_ref,
               out_ref):
    f32 = jnp.float32

    def mm(a, b):
        return jnp.dot(a, b, preferred_element_type=f32)

    def gates(zcat, c_sum):
        i_g = jax.nn.sigmoid(zcat[:, :H])
        o_g = jax.nn.sigmoid(zcat[:, H:2 * H])
        u_g = jnp.tanh(zcat[:, 2 * H:])
        c = i_g * u_g + c_sum
        h = o_g * jnp.tanh(c)
        return h, c

    n = 2 ** (depth - 1)
    xf = nf_ref[n - 1:2 * n - 1].reshape(n * B, D)
    h, c = gates(mm(xf, Wxcat_ref[...]) + bcat_ref[...], 0.0)

    watt = watt_ref[...].reshape(1, 1, H)

    for l in range(depth - 2, -1, -1):
        n = 2 ** l
        m = 2 * n
        c0 = 2 * n - 1

        h3 = h.reshape(m, B, H)
        c3 = c.reshape(m, B, H)

        rel = rel_ref[c0:c0 + m]
        masks = [(rel == r).astype(f32) for r in range(R)]
        remb = masks[0] * rel_emb_ref[0:1, :].reshape(1, 1, H)
        bfs = masks[0] * bf_ref[0:1, :].reshape(1, 1, H)
        for r in range(1, R):
            remb = remb + masks[r] * rel_emb_ref[r:r + 1, :].reshape(1, 1, H)
            bfs = bfs + masks[r] * bf_ref[r:r + 1, :].reshape(1, 1, H)

        s = jnp.sum((h3 + remb) * watt, axis=-1, keepdims=True)
        s4 = s.reshape(n, 2, B, 1)
        a = jax.nn.sigmoid(s4[:, 0] - s4[:, 1])
        h4 = h3.reshape(n, 2, B, H)
        h_sum = a * h4[:, 0] + (1.0 - a) * h4[:, 1]

        fcat = mm(h, Wfcat_ref[...])
        f = jnp.broadcast_to(bfs, (m, B, H))
        for r in range(R):
            f = f + masks[r] * fcat[:, r * H:(r + 1) * H].reshape(m, B, H)
        fc = (f * c3).reshape(n, 2, B, H)
        c_sum = (fc[:, 0] + fc[:, 1]).reshape(n * B, H)

        xf = nf_ref[n - 1:2 * n - 1].reshape(n * B, D)
        hs = h_sum.reshape(n * B, H)
        h, c = gates(mm(xf, Wxcat_ref[...]) + mm(hs, Whcat_ref[...])
                     + bcat_ref[...], c_sum)

    out_ref[...] = h


def kernel(node_features, rel_emb, W_att, b_att, W_i, b_i, W_o, b_o,
           W_u, b_u, Wf, bf, W_enc, b_enc, relation_ids):
    B, N, D = node_features.shape
    R, H = rel_emb.shape
    depth = (N + 1).bit_length() - 1

    nf = jnp.transpose(node_features, (1, 0, 2))
    relH = jnp.broadcast_to(
        relation_ids.astype(jnp.int32)[:, None, None], (N, 1, H))
    watt = W_att.reshape(1, H)
    Wxcat = jnp.concatenate([W_i[:D], W_o[:D], W_u[:D]], axis=1)
    Whcat = jnp.concatenate([W_i[D:], W_o[D:], W_u[D:]], axis=1)
    bcat = jnp.concatenate([b_i, b_o, b_u]).reshape(1, 3 * H)
    Wfcat = jnp.transpose(Wf, (1, 0, 2)).reshape(H, R * H)

    body = functools.partial(_tree_body, depth, B, D, H, R)
    return pl.pallas_call(
        body,
        out_shape=jax.ShapeDtypeStruct((B, H), jnp.float32),
    )(nf, relH, rel_emb, watt, Wfcat, bf, Wxcat, Whcat, bcat)


# speedup vs baseline: 1.2209x; 1.0415x over previous
import functools

import jax
import jax.numpy as jnp
from jax.experimental import pallas as pl


def _tree_body(depth, B, D, H, R,
               nf_ref, rel_ref, rel_emb_ref, watt_ref, Wfcat_ref, bf_ref,
               Wxcat_ref, Whcat_ref, bcat_ref,
               out_ref):
    f32 = jnp.float32

    def mm(a, b):
        return jnp.dot(a, b, preferred_element_type=f32)

    def gates(zcat, c_sum):
        i_g = jax.nn.sigmoid(zcat[:, :H])
        o_g = jax.nn.sigmoid(zcat[:, H:2 * H])
        u_g = jnp.tanh(zcat[:, 2 * H:])
        c = i_g * u_g + c_sum
        h = o_g * jnp.tanh(c)
        return h, c

    n = 2 ** (depth - 1)
    xf = nf_ref[n - 1:2 * n - 1].reshape(n * B, D)
    h, c = gates(mm(xf, Wxcat_ref[...]) + bcat_ref[...], 0.0)

    watt = watt_ref[...].reshape(1, 1, H)

    for l in range(depth - 2, -1, -1):
        n = 2 ** l
        m = 2 * n
        c0 = 2 * n - 1

        h3 = h.reshape(m, B, H)
        c3 = c.reshape(m, B, H)

        rel = rel_ref[c0:c0 + m]

        def sel(rows_ref):
            out = rows_ref[R - 1:R, :].reshape(1, 1, H)
            out = jnp.broadcast_to(out, (m, 1, H))
            for r in range(R - 2, -1, -1):
                out = jnp.where(rel == r,
                                rows_ref[r:r + 1, :].reshape(1, 1, H), out)
            return out

        remb = sel(rel_emb_ref)
        bfs = sel(bf_ref)

        s = (jnp.sum(h3 * watt, axis=-1, keepdims=True)
             + jnp.sum(remb * watt, axis=-1, keepdims=True))
        s4 = s.reshape(n, 2, B, 1)
        a = jax.nn.sigmoid(s4[:, 0] - s4[:, 1])
        h4 = h3.reshape(n, 2, B, H)
        h_sum = h4[:, 1] + a * (h4[:, 0] - h4[:, 1])

        fcat = mm(h, Wfcat_ref[...])
        fs = [fcat[:, r * H:(r + 1) * H].reshape(m, B, H) for r in range(R)]
        f = fs[R - 1]
        for r in range(R - 2, -1, -1):
            f = jnp.where(rel == r, fs[r], f)
        f = f + bfs
        fc = (f * c3).reshape(n, 2, B, H)
        c_sum = (fc[:, 0] + fc[:, 1]).reshape(n * B, H)

        xf = nf_ref[n - 1:2 * n - 1].reshape(n * B, D)
        hs = h_sum.reshape(n * B, H)
        h, c = gates(mm(xf, Wxcat_ref[...]) + mm(hs, Whcat_ref[...])
                     + bcat_ref[...], c_sum)

    out_ref[...] = h


def kernel(node_features, rel_emb, W_att, b_att, W_i, b_i, W_o, b_o,
           W_u, b_u, Wf, bf, W_enc, b_enc, relation_ids):
    B, N, D = node_features.shape
    R, H = rel_emb.shape
    depth = (N + 1).bit_length() - 1

    nf = jnp.transpose(node_features, (1, 0, 2))
    relH = jnp.broadcast_to(
        relation_ids.astype(jnp.int32)[:, None, None], (N, 1, H))
    watt = W_att.reshape(1, H)
    Wxcat = jnp.concatenate([W_i[:D], W_o[:D], W_u[:D]], axis=1)
    Whcat = jnp.concatenate([W_i[D:], W_o[D:], W_u[D:]], axis=1)
    bcat = jnp.concatenate([b_i, b_o, b_u]).reshape(1, 3 * H)
    Wfcat = jnp.transpose(Wf, (1, 0, 2)).reshape(H, R * H)

    body = functools.partial(_tree_body, depth, B, D, H, R)
    return pl.pallas_call(
        body,
        out_shape=jax.ShapeDtypeStruct((B, H), jnp.float32),
    )(nf, relH, rel_emb, watt, Wfcat, bf, Wxcat, Whcat, bcat)
